# baseline (speedup 1.0000x reference)
import functools

import jax
import jax.numpy as jnp
from jax import lax
from jax.experimental import pallas as pl
from jax.experimental.pallas import tpu as pltpu
from jax.experimental.pallas import tpu_sc as plsc

_DIM = 64
_L = 16
_ND = _DIM // _L
_BATCH = 16384
_NC, _NS = 2, 16
_NW = _NC * _NS
_PW = _BATCH // _NW
_CH = 128
_NCHUNK = _PW // _CH
_EPS = 1e-9
_MARGIN = 1.0


def _tec_body(ph, pr, pt, nh, nr, nt, ent_e, ent_c, rel_e, rel_c, out,
              idx_v, rows_v, out_v, sem):
    wid = lax.axis_index("s") * _NC + lax.axis_index("c")

    pltpu.sync_copy(ph.at[wid], idx_v.at[0])
    pltpu.sync_copy(pr.at[wid], idx_v.at[1])
    pltpu.sync_copy(pt.at[wid], idx_v.at[2])
    pltpu.sync_copy(nh.at[wid], idx_v.at[3])
    pltpu.sync_copy(nr.at[wid], idx_v.at[4])
    pltpu.sync_copy(nt.at[wid], idx_v.at[5])

    def issue(c):
        hnd = []
        for base, (h_s, r_s, t_s) in ((0, (0, 1, 2)), (6, (3, 4, 5))):
            hnd.append(pltpu.async_copy(
                ent_e.at[idx_v.at[h_s, c]], rows_v.at[base + 0], sem))
            hnd.append(pltpu.async_copy(
                ent_c.at[idx_v.at[h_s, c]], rows_v.at[base + 1], sem))
            hnd.append(pltpu.async_copy(
                ent_e.at[idx_v.at[t_s, c]], rows_v.at[base + 2], sem))
            hnd.append(pltpu.async_copy(
                ent_c.at[idx_v.at[t_s, c]], rows_v.at[base + 3], sem))
            hnd.append(pltpu.async_copy(
                rel_e.at[idx_v.at[r_s, c]], rows_v.at[base + 4], sem))
            hnd.append(pltpu.async_copy(
                rel_c.at[idx_v.at[r_s, c]], rows_v.at[base + 5], sem))
        return hnd

    lane = lax.iota(jnp.int32, _L)
    perms = [lane ^ k for k in (8, 4, 2, 1)]
    _dn = lax.GatherDimensionNumbers(
        offset_dims=(), collapsed_slice_dims=(0,), start_index_map=(0,))

    def _lane_perm(x, p):
        return lax.gather(
            x, p[:, None], _dn, slice_sizes=(1,),
            mode=lax.GatherScatterMode.PROMISE_IN_BOUNDS)

    def pos_body(i, tot):
        accs = []
        for base in (0, 6):
            acc = jnp.zeros((_L,), jnp.float32)
            for d in range(_ND):
                sl = pl.ds(d * _L, _L)
                hm = rows_v[base + 0, i, sl]
                hv = rows_v[base + 1, i, sl]
                tm = rows_v[base + 2, i, sl]
                tv = rows_v[base + 3, i, sl]
                rm = rows_v[base + 4, i, sl]
                rv = rows_v[base + 5, i, sl]
                em = tm - hm
                ev = tv + hv
                rvp = rv + _EPS
                evp = ev + _EPS
                dmr = rm - em
                sq = dmr * dmr
                num = (ev + sq) * evp + (rv + sq) * rvp
                acc = acc + num / (rvp * evp)
            accs.append(acc)
        diff = accs[0] - accs[1]
        for p in perms:
            diff = diff + _lane_perm(diff, p)
        s = diff * 0.25 + _MARGIN
        return tot + jnp.maximum(s, 0.0)

    total = jnp.zeros((_L,), jnp.float32)
    for c in range(_NCHUNK):
        for h in issue(c):
            h.wait()
        total = lax.fori_loop(0, _CH, pos_body, total)

    out_v[...] = total
    pltpu.sync_copy(out_v, out.at[wid])


@functools.partial(
    pl.kernel,
    out_type=jax.ShapeDtypeStruct((_NW, _L), jnp.float32),
    mesh=plsc.VectorSubcoreMesh(core_axis_name="c", subcore_axis_name="s"),
    compiler_params=pltpu.CompilerParams(use_tc_tiling_on_sc=False),
    scratch_types=[
        pltpu.VMEM((6, _NCHUNK, _CH), jnp.int32),
        pltpu.VMEM((12, _CH, _DIM), jnp.float32),
        pltpu.VMEM((_L,), jnp.float32),
        pltpu.SemaphoreType.DMA,
    ],
)
def _kg2e_sc(*refs):
    _tec_body(*refs)


def kernel(posX, negX, entE, entC, relE, relC):
    shp = (_NW, _NCHUNK, _CH)
    ph = posX[:, 0].reshape(shp)
    pr = posX[:, 1].reshape(shp)
    pt = posX[:, 2].reshape(shp)
    nh = negX[:, 0].reshape(shp)
    nr = negX[:, 1].reshape(shp)
    nt = negX[:, 2].reshape(shp)
    partials = _kg2e_sc(ph, pr, pt, nh, nr, nt, entE, entC, relE, relC)
    return jnp.sum(partials[:, 0]) / _BATCH


# speedup vs baseline: 1.4063x; 1.4063x over previous
import functools

import jax
import jax.numpy as jnp
from jax import lax
from jax.experimental import pallas as pl
from jax.experimental.pallas import tpu as pltpu
from jax.experimental.pallas import tpu_sc as plsc

_DIM = 64
_L = 16
_ND = _DIM // _L
_BATCH = 16384
_NC, _NS = 2, 16
_NW = _NC * _NS
_PW = _BATCH // _NW
_G = 16
_NG = _PW // _G
_EPS = 1e-9
_MARGIN = 1.0


def _tec_body(ph, pr, pt, nh, nr, nt, ent_e, ent_c, rel_e, rel_c, out,
              idx_v, rows_v, out_v, sems):
    wid = lax.axis_index("s") * _NC + lax.axis_index("c")
    base = wid * _PW
    for k, src in enumerate((ph, pr, pt, nh, nr, nt)):
        pltpu.sync_copy(src.at[pl.ds(base, _PW)],
                        idx_v.at[pl.ds(k * _PW, _PW)])

    def issue(g):
        handles = []
        vecs = [idx_v[pl.ds(k * _PW + g * _G, _G)] for k in range(6)]
        for j in range(_G):
            for b, (hk, rk, tk) in ((0, (0, 1, 2)), (6, (3, 4, 5))):
                h = vecs[hk][j]
                r = vecs[rk][j]
                t = vecs[tk][j]
                handles += [
                    pltpu.async_copy(ent_e.at[h], rows_v.at[j, b + 0], sems),
                    pltpu.async_copy(ent_c.at[h], rows_v.at[j, b + 1], sems),
                    pltpu.async_copy(ent_e.at[t], rows_v.at[j, b + 2], sems),
                    pltpu.async_copy(ent_c.at[t], rows_v.at[j, b + 3], sems),
                    pltpu.async_copy(rel_e.at[r], rows_v.at[j, b + 4], sems),
                    pltpu.async_copy(rel_c.at[r], rows_v.at[j, b + 5], sems),
                ]
        return handles

    lane = lax.iota(jnp.int32, _L)
    perms = [lane ^ k for k in (8, 4, 2, 1)]
    _dn = lax.GatherDimensionNumbers(
        offset_dims=(), collapsed_slice_dims=(0,), start_index_map=(0,))

    def _lane_perm(x, perm):
        return lax.gather(
            x, perm[:, None], _dn, slice_sizes=(1,),
            mode=lax.GatherScatterMode.PROMISE_IN_BOUNDS)

    def score(j):
        accs = []
        for b in (0, 6):
            acc = jnp.zeros((_L,), jnp.float32)
            for d in range(_ND):
                sl = pl.ds(d * _L, _L)
                hm = rows_v[j, b + 0, sl]
                hv = rows_v[j, b + 1, sl]
                tm = rows_v[j, b + 2, sl]
                tv = rows_v[j, b + 3, sl]
                rm = rows_v[j, b + 4, sl]
                rv = rows_v[j, b + 5, sl]
                em = tm - hm
                ev = tv + hv
                rvp = rv + _EPS
                evp = ev + _EPS
                dmr = rm - em
                sq = dmr * dmr
                num = (ev + sq) * evp + (rv + sq) * rvp
                acc = acc + num / (rvp * evp)
            accs.append(acc)
        diff = accs[0] - accs[1]
        for perm in perms:
            diff = diff + _lane_perm(diff, perm)
        return jnp.maximum(diff * 0.25 + _MARGIN, 0.0)

    def body(g, tot):
        for h in issue(g):
            h.wait()
        for j in range(_G):
            tot = tot + score(j)
        return tot

    total = lax.fori_loop(0, _NG, body, jnp.zeros((_L,), jnp.float32))

    out_v[pl.ds(0, _L)] = total
    pltpu.sync_copy(out_v, out.at[pl.ds(wid * 128, 128)])


@functools.partial(
    pl.kernel,
    out_type=jax.ShapeDtypeStruct((_NW * 128,), jnp.float32),
    mesh=plsc.VectorSubcoreMesh(core_axis_name="c", subcore_axis_name="s"),
    compiler_params=pltpu.CompilerParams(use_tc_tiling_on_sc=True),
    scratch_types=[
        pltpu.VMEM((6 * _PW,), jnp.int32),
        pltpu.VMEM((_G, 12, _DIM), jnp.float32),
        pltpu.VMEM((128,), jnp.float32),
        pltpu.SemaphoreType.DMA,
    ],
)
def _kg2e_sc(*refs):
    _tec_body(*refs)


def kernel(posX, negX, entE, entC, relE, relC):
    ph = posX[:, 0]
    pr = posX[:, 1]
    pt = posX[:, 2]
    nh = negX[:, 0]
    nr = negX[:, 1]
    nt = negX[:, 2]
    partials = _kg2e_sc(ph, pr, pt, nh, nr, nt, entE, entC, relE, relC)
    return jnp.sum(partials.reshape(_NW, 128)[:, 0]) / _BATCH
